# baseline (speedup 1.0000x reference)
import jax
import jax.numpy as jnp
from jax.experimental import pallas as pl
from jax.experimental.pallas import tpu as pltpu

_N = 16384
_C_IN = 128
_C_OUT = 16
_EPS = 1e-5
_BM = 128
_NI = _N // _BM


def _simplicial_conv_kernel(zh_ref, l1_ref, wt_ref, b_ref, g_ref, beta_ref,
                            out_ref, ztheta_ref, zconvT_ref):
    i = pl.program_id(0)

    @pl.when(i == 0)
    def _compute_ztheta():
        ztheta_ref[...] = (
            jnp.dot(zh_ref[...], wt_ref[...],
                    preferred_element_type=jnp.float32)
            + b_ref[...]
        )

    blk = jnp.dot(l1_ref[...], ztheta_ref[...],
                  preferred_element_type=jnp.float32)
    zconvT_ref[:, pl.ds(i * _BM, _BM)] = blk.T

    @pl.when(i == _NI - 1)
    def _epilogue():
        zc = zconvT_ref[...]
        mean = jnp.mean(zc, axis=1, keepdims=True)
        centered = zc - mean
        var = jnp.mean(centered * centered, axis=1, keepdims=True)
        inv = jax.lax.rsqrt(var + _EPS)
        zn = centered * inv
        zp = jnp.maximum(g_ref[...] * zn + beta_ref[...], 0.0)
        out_ref[...] = jnp.max(zp, axis=0, keepdims=True)


def kernel(Z_H, L1_tilde, W, b, gamma, beta):
    Wt = W.T
    b_row = b.reshape(1, _C_OUT)
    g_col = gamma.reshape(_C_OUT, 1)
    beta_col = beta.reshape(_C_OUT, 1)

    out = pl.pallas_call(
        _simplicial_conv_kernel,
        grid=(_NI,),
        in_specs=[
            pl.BlockSpec((_N, _C_IN), lambda i: (0, 0)),
            pl.BlockSpec((_BM, _N), lambda i: (i, 0)),
            pl.BlockSpec((_C_IN, _C_OUT), lambda i: (0, 0)),
            pl.BlockSpec((1, _C_OUT), lambda i: (0, 0)),
            pl.BlockSpec((_C_OUT, 1), lambda i: (0, 0)),
            pl.BlockSpec((_C_OUT, 1), lambda i: (0, 0)),
        ],
        out_specs=pl.BlockSpec((1, _N), lambda i: (0, 0)),
        out_shape=jax.ShapeDtypeStruct((1, _N), jnp.float32),
        scratch_shapes=[
            pltpu.VMEM((_N, _C_OUT), jnp.float32),
            pltpu.VMEM((_C_OUT, _N), jnp.float32),
        ],
    )(Z_H, L1_tilde, Wt, b_row, g_col, beta_col)
    return out.reshape(_N, 1)


# speedup vs baseline: 1.0124x; 1.0124x over previous
import jax
import jax.numpy as jnp
from jax.experimental import pallas as pl
from jax.experimental.pallas import tpu as pltpu

_N = 16384
_C_IN = 128
_C_OUT = 16
_EPS = 1e-5
_BM = 256
_NI = _N // _BM


def _simplicial_conv_kernel(zh_ref, l1_ref, wt_ref, b_ref, g_ref, beta_ref,
                            out_ref, ztheta_ref, zconvT_ref):
    i = pl.program_id(0)

    @pl.when(i == 0)
    def _compute_ztheta():
        ztheta_ref[...] = (
            jnp.dot(zh_ref[...], wt_ref[...],
                    preferred_element_type=jnp.float32)
            + b_ref[...]
        )

    blk = jnp.dot(l1_ref[...], ztheta_ref[...],
                  preferred_element_type=jnp.float32)
    zconvT_ref[:, pl.ds(i * _BM, _BM)] = blk.T

    @pl.when(i == _NI - 1)
    def _epilogue():
        zc = zconvT_ref[...]
        mean = jnp.mean(zc, axis=1, keepdims=True)
        centered = zc - mean
        var = jnp.mean(centered * centered, axis=1, keepdims=True)
        inv = jax.lax.rsqrt(var + _EPS)
        zn = centered * inv
        zp = jnp.maximum(g_ref[...] * zn + beta_ref[...], 0.0)
        out_ref[...] = jnp.max(zp, axis=0, keepdims=True)


def kernel(Z_H, L1_tilde, W, b, gamma, beta):
    Wt = W.T
    b_row = b.reshape(1, _C_OUT)
    g_col = gamma.reshape(_C_OUT, 1)
    beta_col = beta.reshape(_C_OUT, 1)

    out = pl.pallas_call(
        _simplicial_conv_kernel,
        grid=(_NI,),
        in_specs=[
            pl.BlockSpec((_N, _C_IN), lambda i: (0, 0)),
            pl.BlockSpec((_BM, _N), lambda i: (i, 0)),
            pl.BlockSpec((_C_IN, _C_OUT), lambda i: (0, 0)),
            pl.BlockSpec((1, _C_OUT), lambda i: (0, 0)),
            pl.BlockSpec((_C_OUT, 1), lambda i: (0, 0)),
            pl.BlockSpec((_C_OUT, 1), lambda i: (0, 0)),
        ],
        out_specs=pl.BlockSpec((1, _N), lambda i: (0, 0)),
        out_shape=jax.ShapeDtypeStruct((1, _N), jnp.float32),
        scratch_shapes=[
            pltpu.VMEM((_N, _C_OUT), jnp.float32),
            pltpu.VMEM((_C_OUT, _N), jnp.float32),
        ],
    )(Z_H, L1_tilde, Wt, b_row, g_col, beta_col)
    return out.reshape(_N, 1)
